# baseline (speedup 1.0000x reference)
import functools

import jax
import jax.numpy as jnp
from jax import lax
from jax.experimental import pallas as pl
from jax.experimental.pallas import tpu as pltpu
from jax.experimental.pallas import tpu_sc as plsc

B = 4096
F = 26
K = 32
N = B * F
NC = 2
NS = 16
NW = NC * NS
PER_W = N // NW
CHUNK = 128
NCH = PER_W // CHUNK


def _fm_sc(idx_hbm, val_hbm, table_hbm, out_hbm, idx_v, val_v, rows_v, sem):
    wid = lax.axis_index("s") * NC + lax.axis_index("c")
    base = wid * PER_W
    pltpu.sync_copy(idx_hbm.at[pl.ds(base, PER_W)], idx_v)
    pltpu.sync_copy(val_hbm.at[pl.ds(base, PER_W)], val_v)

    copies = []
    for j in range(NCH):
        copies.append(
            pltpu.async_copy(
                table_hbm.at[idx_v.at[pl.ds(j * CHUNK, CHUNK)]],
                rows_v.at[pl.ds(j * CHUNK, CHUNK)],
                sem,
            )
        )
    for c in copies:
        c.wait()

    def body(g, _):
        vv = val_v[pl.ds(g * 16, 16)]
        for j in range(16):
            r = g * 16 + j
            v = vv[j]
            rows_v[r, pl.ds(0, 16)] = rows_v[r, pl.ds(0, 16)] * v
            rows_v[r, pl.ds(16, 16)] = rows_v[r, pl.ds(16, 16)] * v
        return _

    lax.fori_loop(0, PER_W // 16, body, 0)

    pltpu.sync_copy(rows_v, out_hbm.at[pl.ds(base, PER_W)])


@jax.jit
def _fm(idx_flat, val_flat, table):
    mesh = plsc.VectorSubcoreMesh(core_axis_name="c", subcore_axis_name="s")
    run = functools.partial(
        pl.kernel,
        mesh=mesh,
        out_type=jax.ShapeDtypeStruct((N, K), jnp.float32),
        scratch_types=[
            pltpu.VMEM((PER_W,), jnp.int32),
            pltpu.VMEM((PER_W,), jnp.float32),
            pltpu.VMEM((PER_W, K), jnp.float32),
            pltpu.SemaphoreType.DMA,
        ],
        compiler_params=pltpu.CompilerParams(use_tc_tiling_on_sc=False),
    )(_fm_sc)
    return run(idx_flat, val_flat, table)


def kernel(nonzero_index, nonzero_value, table):
    idx_flat = nonzero_index.reshape(N).astype(jnp.int32)
    val_flat = nonzero_value.reshape(N)
    out = _fm(idx_flat, val_flat, table)
    return out.reshape(B, F, K)
